# Initial kernel scaffold; baseline (speedup 1.0000x reference)
import jax
import jax.numpy as jnp
from jax.experimental import pallas as pl


def kernel(x, edge_index, batch, W1, b1, W2, b2, W3, b3, l1_W, l1_b, l2_W, l2_b):
    raise NotImplementedError("write your pallas kernel here")


# speedup vs baseline: 6.9743x; 6.9743x over previous
import functools

import jax
import jax.numpy as jnp
from jax import lax
from jax.experimental import pallas as pl
from jax.experimental.pallas import tpu as pltpu
from jax.experimental.pallas import tpu_sc as plsc

N = 50000
E = 800000
G = 512
D_IN = 100

STRIPE_N = 3136
NA = 16 * STRIPE_N
ER = 6400
EA = ER * 128
BR = 416
NBP = BR * 128
STRIPE_G = 40
GA = 16 * STRIPE_G


def _sc_mesh():
    return plsc.VectorSubcoreMesh(core_axis_name="c", subcore_axis_name="s",
                                  num_cores=2, num_subcores=16)


@functools.cache
def _sc_gather_scatter(n_rows, dh, nt, nacc, stripe, qbase=0):
    rpt = n_rows // 16

    @functools.partial(
        pl.kernel,
        mesh=_sc_mesh(),
        compiler_params=pltpu.CompilerParams(use_tc_tiling_on_sc=False),
        out_type=jax.ShapeDtypeStruct((2 * nacc, dh), jnp.float32),
        scratch_types=[
            pltpu.VMEM((128,), jnp.int32),
            pltpu.VMEM((128,), jnp.int32),
            pltpu.VMEM((128, dh), jnp.float32),
            pltpu.VMEM((stripe, dh), jnp.float32),
            pltpu.VMEM_SHARED((nacc, dh), jnp.float32),
            pltpu.SemaphoreType.DMA,
        ],
    )
    def k(gidx_hbm, sidx_hbm, table_hbm, zeros_hbm, out_hbm,
          gidx_v, sidx_v, rows_v, stripe_v, acc, sem):
        c = lax.axis_index("c")
        sid = lax.axis_index("s")
        pltpu.sync_copy(zeros_hbm.at[pl.ds(sid * stripe, stripe)], stripe_v)
        pltpu.sync_copy(stripe_v, acc.at[pl.ds(sid * stripe, stripe)])
        plsc.subcore_barrier()
        base = sid * rpt
        off = (qbase + c) * nt

        def body(r, carry):
            row = base + r
            pltpu.sync_copy(gidx_hbm.at[row], gidx_v)
            pltpu.sync_copy(sidx_hbm.at[row], sidx_v)
            for j in range(8):
                sl = pl.ds(j * 16, 16)
                gidx_v[sl] = gidx_v[sl] + off
            pltpu.async_copy(table_hbm.at[gidx_v], rows_v, sem).wait()
            pltpu.sync_copy(rows_v, acc.at[sidx_v], add=True)
            return carry

        lax.fori_loop(0, rpt, body, 0)
        plsc.subcore_barrier()
        pltpu.sync_copy(acc.at[pl.ds(sid * stripe, stripe)], stripe_v)
        pltpu.sync_copy(stripe_v,
                        out_hbm.at[pl.ds(c * nacc + sid * stripe, stripe)])

    return k


@functools.cache
def _sc_degree_count():
    ept = ER // 32
    bpt = BR // 32

    @functools.partial(
        pl.kernel,
        mesh=_sc_mesh(),
        compiler_params=pltpu.CompilerParams(use_tc_tiling_on_sc=False),
        out_type=(jax.ShapeDtypeStruct((2 * NA,), jnp.float32),
                  jax.ShapeDtypeStruct((2 * GA,), jnp.float32)),
        scratch_types=[
            pltpu.VMEM((128,), jnp.int32),
            pltpu.VMEM((128,), jnp.float32),
            pltpu.VMEM((STRIPE_N,), jnp.float32),
            pltpu.VMEM((STRIPE_G,), jnp.float32),
            pltpu.VMEM_SHARED((NA,), jnp.float32),
            pltpu.VMEM_SHARED((GA,), jnp.float32),
        ],
    )
    def k(dst_hbm, bat_hbm, zn_hbm, zg_hbm, deg_hbm, cnt_hbm,
          idx_v, ones_v, sn_v, sg_v, dacc, cacc):
        c = lax.axis_index("c")
        sid = lax.axis_index("s")
        w = c * 16 + sid
        pltpu.sync_copy(zn_hbm.at[pl.ds(sid * STRIPE_N, STRIPE_N)], sn_v)
        pltpu.sync_copy(sn_v, dacc.at[pl.ds(sid * STRIPE_N, STRIPE_N)])
        pltpu.sync_copy(zg_hbm.at[pl.ds(sid * STRIPE_G, STRIPE_G)], sg_v)
        pltpu.sync_copy(sg_v, cacc.at[pl.ds(sid * STRIPE_G, STRIPE_G)])
        for j in range(8):
            ones_v[pl.ds(j * 16, 16)] = jnp.ones((16,), jnp.float32)
        plsc.subcore_barrier()

        ebase = w * ept

        def ebody(r, carry):
            pltpu.sync_copy(dst_hbm.at[ebase + r], idx_v)
            pltpu.sync_copy(ones_v, dacc.at[idx_v], add=True)
            return carry

        lax.fori_loop(0, ept, ebody, 0)

        bbase = w * bpt

        def bbody(r, carry):
            pltpu.sync_copy(bat_hbm.at[bbase + r], idx_v)
            pltpu.sync_copy(ones_v, cacc.at[idx_v], add=True)
            return carry

        lax.fori_loop(0, bpt, bbody, 0)
        plsc.subcore_barrier()
        pltpu.sync_copy(dacc.at[pl.ds(sid * STRIPE_N, STRIPE_N)], sn_v)
        pltpu.sync_copy(
            sn_v, deg_hbm.at[pl.ds(c * NA + sid * STRIPE_N, STRIPE_N)])
        pltpu.sync_copy(cacc.at[pl.ds(sid * STRIPE_G, STRIPE_G)], sg_v)
        pltpu.sync_copy(
            sg_v, cnt_hbm.at[pl.ds(c * GA + sid * STRIPE_G, STRIPE_G)])

    return k


def _mm1_body(x_ref, w_ref, o_ref):
    o_ref[0] = jnp.dot(x_ref[...], w_ref[0],
                       preferred_element_type=jnp.float32)


_TC_MM1 = pl.pallas_call(
    _mm1_body,
    grid=(4, 16),
    in_specs=[pl.BlockSpec((STRIPE_N, D_IN), lambda h, i: (i, 0)),
              pl.BlockSpec((1, D_IN, 16), lambda h, i: (h, 0, 0))],
    out_specs=pl.BlockSpec((1, STRIPE_N, 16), lambda h, i: (h, i, 0)),
    out_shape=jax.ShapeDtypeStruct((4, NA, 16), jnp.float32),
)


def _tc2_body(d0_ref, d1_ref, t1h_ref, tp_ref, dv_ref):
    dv = lax.rsqrt(d0_ref[...] + d1_ref[...] + 1.0)
    dv_ref[...] = dv
    tp_ref[0] = t1h_ref[0] * dv


_TC_SCALE = pl.pallas_call(
    _tc2_body,
    grid=(4, 16),
    in_specs=[pl.BlockSpec((STRIPE_N, 1), lambda h, i: (i, 0)),
              pl.BlockSpec((STRIPE_N, 1), lambda h, i: (i, 0)),
              pl.BlockSpec((1, STRIPE_N, 16), lambda h, i: (h, i, 0))],
    out_specs=[pl.BlockSpec((1, STRIPE_N, 16), lambda h, i: (h, i, 0)),
               pl.BlockSpec((STRIPE_N, 1), lambda h, i: (i, 0))],
    out_shape=[jax.ShapeDtypeStruct((4, NA, 16), jnp.float32),
               jax.ShapeDtypeStruct((NA, 1), jnp.float32)],
)


def _layer_body(*refs):
    n_s = (len(refs) - 5) // 2
    s_refs = refs[:n_s]
    t_refs = refs[n_s:2 * n_s]
    dv_ref, b_ref, w_ref, tp_ref, tr_ref = refs[2 * n_s:]
    dv = dv_ref[...]
    s = jnp.concatenate([r[0] for r in s_refs], axis=1)
    t = jnp.concatenate([r[0] for r in t_refs], axis=1)
    hcur = jnp.maximum(s * dv + t * (dv * dv) + b_ref[...], 0.0)
    th = jnp.dot(hcur, w_ref[0], preferred_element_type=jnp.float32)
    tr_ref[0] = th
    tp_ref[0] = th * dv


def _tc_layer(din, dout, n_parts):
    dh_in = din // n_parts
    dh_out = dout // 2
    part = lambda q: pl.BlockSpec((1, STRIPE_N, dh_in),
                                  lambda h, i, q=q: (q, i, 0))
    return pl.pallas_call(
        _layer_body,
        grid=(2, 16),
        in_specs=([part(q) for q in range(n_parts)]
                  + [part(q) for q in range(n_parts)]
                  + [pl.BlockSpec((STRIPE_N, 1), lambda h, i: (i, 0)),
                     pl.BlockSpec((1, din), lambda h, i: (0, 0)),
                     pl.BlockSpec((1, din, dh_out), lambda h, i: (h, 0, 0))]),
        out_specs=[pl.BlockSpec((1, STRIPE_N, dh_out), lambda h, i: (h, i, 0)),
                   pl.BlockSpec((1, STRIPE_N, dh_out), lambda h, i: (h, i, 0))],
        out_shape=[jax.ShapeDtypeStruct((2, NA, dh_out), jnp.float32),
                   jax.ShapeDtypeStruct((2, NA, dh_out), jnp.float32)],
    )


_TC_L1 = _tc_layer(64, 32, 4)
_TC_L2 = _tc_layer(32, 16, 2)


def _tc5_body(sh_ref, t3h_ref, dv_ref, b3h_ref, hp_ref):
    i = pl.program_id(1)
    dv = dv_ref[...]
    rid = i * STRIPE_N + lax.broadcasted_iota(jnp.int32, (STRIPE_N, 1), 0)
    valid = (rid < N).astype(jnp.float32)
    hp_ref[0] = (sh_ref[0] * dv + t3h_ref[0] * (dv * dv)
                 + b3h_ref[0]) * valid


_TC_H3 = pl.pallas_call(
    _tc5_body,
    grid=(2, 16),
    in_specs=[pl.BlockSpec((1, STRIPE_N, 8), lambda h, i: (h, i, 0)),
              pl.BlockSpec((1, STRIPE_N, 8), lambda h, i: (h, i, 0)),
              pl.BlockSpec((STRIPE_N, 1), lambda h, i: (i, 0)),
              pl.BlockSpec((1, 1, 8), lambda h, i: (h, 0, 0))],
    out_specs=pl.BlockSpec((1, STRIPE_N, 8), lambda h, i: (h, i, 0)),
    out_shape=jax.ShapeDtypeStruct((2, NA, 8), jnp.float32),
)


def _tc6_body(pa_ref, pb_ref, c0_ref, c1_ref, w1_ref, b1_ref, w2_ref, b2_ref,
              o_ref):
    cnt = jnp.maximum(c0_ref[...] + c1_ref[...], 1.0)
    p = jnp.concatenate([pa_ref[...], pb_ref[...]], axis=1) / cnt
    z = jnp.dot(p, w1_ref[...], preferred_element_type=jnp.float32) + b1_ref[...]
    z = jnp.dot(z, w2_ref[...], preferred_element_type=jnp.float32) + b2_ref[...]
    o_ref[...] = 1.0 / (1.0 + jnp.exp(-z))


_TC_HEAD = pl.pallas_call(
    _tc6_body,
    out_shape=jax.ShapeDtypeStruct((G, 2), jnp.float32),
)


@jax.jit
def kernel(x, edge_index, batch, W1, b1, W2, b2, W3, b3, l1_W, l1_b, l2_W, l2_b):
    f32 = jnp.float32
    x_pad = jnp.zeros((NA, D_IN), f32).at[:N].set(x)
    W1p = jnp.zeros((D_IN, 64), f32).at[:, :42].set(W1)
    W1q = jnp.stack([W1p[:, 16 * q:16 * (q + 1)] for q in range(4)])
    b1p = jnp.zeros((1, 64), f32).at[0, :42].set(b1)
    W2p = jnp.zeros((64, 32), f32).at[:42, :24].set(W2)
    W2s = jnp.stack([W2p[:, :16], W2p[:, 16:]])
    b2p = jnp.zeros((1, 32), f32).at[0, :24].set(b2)
    W3p = jnp.zeros((32, 16), f32).at[:24].set(W3)
    W3s = jnp.stack([W3p[:, :8], W3p[:, 8:]])
    b3s = jnp.stack([b3[:8], b3[8:]]).reshape(2, 1, 8)

    src = edge_index[0].astype(jnp.int32)
    dst = edge_index[1].astype(jnp.int32)
    pad_e = jnp.full((EA - E,), N, jnp.int32)
    src_rows = jnp.concatenate([src, pad_e]).reshape(ER, 128)
    dst_rows = jnp.concatenate([dst, pad_e]).reshape(ER, 128)
    bat_rows = jnp.concatenate(
        [batch.astype(jnp.int32), jnp.full((NBP - N,), G, jnp.int32)]
    ).reshape(BR, 128)
    pool_rows = jnp.concatenate(
        [jnp.arange(N, dtype=jnp.int32), jnp.full((NBP - N,), N, jnp.int32)]
    ).reshape(BR, 128)

    deg2, cnt2 = _sc_degree_count()(dst_rows, bat_rows,
                            jnp.zeros((NA,), f32), jnp.zeros((GA,), f32))
    t1 = _TC_MM1(x_pad, W1q)

    d0 = deg2[:NA].reshape(NA, 1)
    d1 = deg2[NA:].reshape(NA, 1)
    tp1, dinv = _TC_SCALE(d0, d1, t1)

    tq1 = tp1.reshape(4 * NA, 16)
    z16 = jnp.zeros((NA, 16), f32)
    sA = _sc_gather_scatter(ER, 16, NA, NA, STRIPE_N, 0)(
        src_rows, dst_rows, tq1, z16)
    sB = _sc_gather_scatter(ER, 16, NA, NA, STRIPE_N, 2)(
        src_rows, dst_rows, tq1, z16)
    s1 = jnp.concatenate([sA, sB]).reshape(4, NA, 16)
    tp2, t2 = _TC_L1(s1, s1, s1, s1, t1, t1, t1, t1, dinv, b1p, W2s)

    s2 = _sc_gather_scatter(ER, 16, NA, NA, STRIPE_N, 0)(
        src_rows, dst_rows, tp2.reshape(2 * NA, 16), z16).reshape(2, NA, 16)
    tp3, t3 = _TC_L2(s2, s2, t2, t2, dinv, b2p, W3s)

    s3 = _sc_gather_scatter(ER, 8, NA, NA, STRIPE_N)(
        src_rows, dst_rows, tp3.reshape(2 * NA, 8),
        jnp.zeros((NA, 8), f32)).reshape(2, NA, 8)
    hp = _TC_H3(s3, t3, dinv, b3s)

    ps = _sc_gather_scatter(BR, 8, NA, GA, STRIPE_G)(
        pool_rows, bat_rows, hp.reshape(2 * NA, 8),
        jnp.zeros((GA, 8), f32))
    pa = ps[:GA][:G]
    pb = ps[GA:][:G]
    c0 = cnt2[:GA][:G].reshape(G, 1)
    c1 = cnt2[GA:][:G].reshape(G, 1)
    return _TC_HEAD(pa, pb, c0, c1, l1_W, l1_b.reshape(1, 10),
                    l2_W, l2_b.reshape(1, 2))


# speedup vs baseline: 12.5684x; 1.8021x over previous
import functools

import jax
import jax.numpy as jnp
from jax import lax
from jax.experimental import pallas as pl
from jax.experimental.pallas import tpu as pltpu
from jax.experimental.pallas import tpu_sc as plsc

N = 50000
E = 800000
G = 512
D_IN = 100

STRIPE_N = 3136
NA = 16 * STRIPE_N
ER = 6400
EA = ER * 128
BR = 416
NBP = BR * 128
STRIPE_G = 40
GA = 16 * STRIPE_G


def _sc_mesh():
    return plsc.VectorSubcoreMesh(core_axis_name="c", subcore_axis_name="s",
                                  num_cores=2, num_subcores=16)


@functools.cache
def _sc_gather_scatter(n_rows, dh, nacc, stripe, qbase, kk):
    rpt = n_rows // 16
    n_sc = rpt // kk

    @functools.partial(
        pl.kernel,
        mesh=_sc_mesh(),
        compiler_params=pltpu.CompilerParams(use_tc_tiling_on_sc=False),
        out_type=jax.ShapeDtypeStruct((2 * nacc, dh), jnp.float32),
        scratch_types=[
            pltpu.VMEM((kk, 128), jnp.int32),
            pltpu.VMEM((kk, 128), jnp.int32),
            pltpu.VMEM((kk, 128, dh), jnp.float32),
            pltpu.VMEM((stripe, dh), jnp.float32),
            pltpu.VMEM_SHARED((nacc, dh), jnp.float32),
            pltpu.SemaphoreType.DMA,
        ],
    )
    def k(gidx_hbm, sidx_hbm, table_hbm, zeros_hbm, out_hbm,
          gidx_v, sidx_v, rows_v, stripe_v, acc, sem):
        c = lax.axis_index("c")
        sid = lax.axis_index("s")
        pltpu.sync_copy(zeros_hbm.at[pl.ds(sid * stripe, stripe)], stripe_v)
        pltpu.sync_copy(stripe_v, acc.at[pl.ds(sid * stripe, stripe)])
        plsc.subcore_barrier()
        gbase = (qbase + c) * n_rows + sid * rpt
        base = sid * rpt

        def body(g, carry):
            pltpu.sync_copy(gidx_hbm.at[pl.ds(gbase + g * kk, kk)], gidx_v)
            pltpu.sync_copy(sidx_hbm.at[pl.ds(base + g * kk, kk)], sidx_v)
            descs = [pltpu.async_copy(table_hbm.at[gidx_v.at[j]],
                                      rows_v.at[j], sem)
                     for j in range(kk)]
            for d in descs:
                d.wait()
            for j in range(kk):
                pltpu.sync_copy(rows_v.at[j], acc.at[sidx_v.at[j]], add=True)
            return carry

        lax.fori_loop(0, n_sc, body, 0)
        plsc.subcore_barrier()
        pltpu.sync_copy(acc.at[pl.ds(sid * stripe, stripe)], stripe_v)
        pltpu.sync_copy(stripe_v,
                        out_hbm.at[pl.ds(c * nacc + sid * stripe, stripe)])

    return k


@functools.cache
def _sc_degree_count():
    ept = ER // 32
    bpt = BR // 32

    @functools.partial(
        pl.kernel,
        mesh=_sc_mesh(),
        compiler_params=pltpu.CompilerParams(use_tc_tiling_on_sc=False),
        out_type=(jax.ShapeDtypeStruct((2 * NA,), jnp.float32),
                  jax.ShapeDtypeStruct((2 * GA,), jnp.float32)),
        scratch_types=[
            pltpu.VMEM((128,), jnp.int32),
            pltpu.VMEM((128,), jnp.float32),
            pltpu.VMEM((STRIPE_N,), jnp.float32),
            pltpu.VMEM((STRIPE_G,), jnp.float32),
            pltpu.VMEM_SHARED((NA,), jnp.float32),
            pltpu.VMEM_SHARED((GA,), jnp.float32),
        ],
    )
    def k(dst_hbm, bat_hbm, zn_hbm, zg_hbm, deg_hbm, cnt_hbm,
          idx_v, ones_v, sn_v, sg_v, dacc, cacc):
        c = lax.axis_index("c")
        sid = lax.axis_index("s")
        w = c * 16 + sid
        pltpu.sync_copy(zn_hbm.at[pl.ds(sid * STRIPE_N, STRIPE_N)], sn_v)
        pltpu.sync_copy(sn_v, dacc.at[pl.ds(sid * STRIPE_N, STRIPE_N)])
        pltpu.sync_copy(zg_hbm.at[pl.ds(sid * STRIPE_G, STRIPE_G)], sg_v)
        pltpu.sync_copy(sg_v, cacc.at[pl.ds(sid * STRIPE_G, STRIPE_G)])
        for j in range(8):
            ones_v[pl.ds(j * 16, 16)] = jnp.ones((16,), jnp.float32)
        plsc.subcore_barrier()

        ebase = w * ept

        def ebody(r, carry):
            pltpu.sync_copy(dst_hbm.at[ebase + r], idx_v)
            pltpu.sync_copy(ones_v, dacc.at[idx_v], add=True)
            return carry

        lax.fori_loop(0, ept, ebody, 0)

        bbase = w * bpt

        def bbody(r, carry):
            pltpu.sync_copy(bat_hbm.at[bbase + r], idx_v)
            pltpu.sync_copy(ones_v, cacc.at[idx_v], add=True)
            return carry

        lax.fori_loop(0, bpt, bbody, 0)
        plsc.subcore_barrier()
        pltpu.sync_copy(dacc.at[pl.ds(sid * STRIPE_N, STRIPE_N)], sn_v)
        pltpu.sync_copy(
            sn_v, deg_hbm.at[pl.ds(c * NA + sid * STRIPE_N, STRIPE_N)])
        pltpu.sync_copy(cacc.at[pl.ds(sid * STRIPE_G, STRIPE_G)], sg_v)
        pltpu.sync_copy(
            sg_v, cnt_hbm.at[pl.ds(c * GA + sid * STRIPE_G, STRIPE_G)])

    return k


def _mm1_body(x_ref, w_ref, o_ref):
    o_ref[0] = jnp.dot(x_ref[...], w_ref[0],
                       preferred_element_type=jnp.float32)


_TC_MM1 = pl.pallas_call(
    _mm1_body,
    grid=(4, 16),
    in_specs=[pl.BlockSpec((STRIPE_N, D_IN), lambda h, i: (i, 0)),
              pl.BlockSpec((1, D_IN, 16), lambda h, i: (h, 0, 0))],
    out_specs=pl.BlockSpec((1, STRIPE_N, 16), lambda h, i: (h, i, 0)),
    out_shape=jax.ShapeDtypeStruct((4, NA, 16), jnp.float32),
)


def _tc2_body(d0_ref, d1_ref, t1h_ref, tp_ref, dv_ref):
    dv = lax.rsqrt(d0_ref[...] + d1_ref[...] + 1.0)
    dv_ref[...] = dv
    tp_ref[0] = t1h_ref[0] * dv


_TC_SCALE = pl.pallas_call(
    _tc2_body,
    grid=(4, 16),
    in_specs=[pl.BlockSpec((STRIPE_N, 1), lambda h, i: (i, 0)),
              pl.BlockSpec((STRIPE_N, 1), lambda h, i: (i, 0)),
              pl.BlockSpec((1, STRIPE_N, 16), lambda h, i: (h, i, 0))],
    out_specs=[pl.BlockSpec((1, STRIPE_N, 16), lambda h, i: (h, i, 0)),
               pl.BlockSpec((STRIPE_N, 1), lambda h, i: (i, 0))],
    out_shape=[jax.ShapeDtypeStruct((4, NA, 16), jnp.float32),
               jax.ShapeDtypeStruct((NA, 1), jnp.float32)],
)


def _layer_body(*refs):
    n_s = (len(refs) - 5) // 2
    s_refs = refs[:n_s]
    t_refs = refs[n_s:2 * n_s]
    dv_ref, b_ref, w_ref, tp_ref, tr_ref = refs[2 * n_s:]
    dv = dv_ref[...]
    s = jnp.concatenate([r[0] for r in s_refs], axis=1)
    t = jnp.concatenate([r[0] for r in t_refs], axis=1)
    hcur = jnp.maximum(s * dv + t * (dv * dv) + b_ref[...], 0.0)
    th = jnp.dot(hcur, w_ref[0], preferred_element_type=jnp.float32)
    tr_ref[0] = th
    tp_ref[0] = th * dv


def _tc_layer(din, dout, n_parts):
    dh_in = din // n_parts
    dh_out = dout // 2
    part = lambda q: pl.BlockSpec((1, STRIPE_N, dh_in),
                                  lambda h, i, q=q: (q, i, 0))
    return pl.pallas_call(
        _layer_body,
        grid=(2, 16),
        in_specs=([part(q) for q in range(n_parts)]
                  + [part(q) for q in range(n_parts)]
                  + [pl.BlockSpec((STRIPE_N, 1), lambda h, i: (i, 0)),
                     pl.BlockSpec((1, din), lambda h, i: (0, 0)),
                     pl.BlockSpec((1, din, dh_out), lambda h, i: (h, 0, 0))]),
        out_specs=[pl.BlockSpec((1, STRIPE_N, dh_out), lambda h, i: (h, i, 0)),
                   pl.BlockSpec((1, STRIPE_N, dh_out), lambda h, i: (h, i, 0))],
        out_shape=[jax.ShapeDtypeStruct((2, NA, dh_out), jnp.float32),
                   jax.ShapeDtypeStruct((2, NA, dh_out), jnp.float32)],
    )


_TC_L1 = _tc_layer(64, 32, 4)
_TC_L2 = _tc_layer(32, 16, 2)


def _tc5_body(sh_ref, t3h_ref, dv_ref, b3h_ref, hp_ref):
    i = pl.program_id(1)
    dv = dv_ref[...]
    rid = i * STRIPE_N + lax.broadcasted_iota(jnp.int32, (STRIPE_N, 1), 0)
    valid = (rid < N).astype(jnp.float32)
    hp_ref[0] = (sh_ref[0] * dv + t3h_ref[0] * (dv * dv)
                 + b3h_ref[0]) * valid


_TC_H3 = pl.pallas_call(
    _tc5_body,
    grid=(2, 16),
    in_specs=[pl.BlockSpec((1, STRIPE_N, 8), lambda h, i: (h, i, 0)),
              pl.BlockSpec((1, STRIPE_N, 8), lambda h, i: (h, i, 0)),
              pl.BlockSpec((STRIPE_N, 1), lambda h, i: (i, 0)),
              pl.BlockSpec((1, 1, 8), lambda h, i: (h, 0, 0))],
    out_specs=pl.BlockSpec((1, STRIPE_N, 8), lambda h, i: (h, i, 0)),
    out_shape=jax.ShapeDtypeStruct((2, NA, 8), jnp.float32),
)


def _tc6_body(pa_ref, pb_ref, c0_ref, c1_ref, w1_ref, b1_ref, w2_ref, b2_ref,
              o_ref):
    cnt = jnp.maximum(c0_ref[...] + c1_ref[...], 1.0)
    p = jnp.concatenate([pa_ref[...], pb_ref[...]], axis=1) / cnt
    z = jnp.dot(p, w1_ref[...], preferred_element_type=jnp.float32) + b1_ref[...]
    z = jnp.dot(z, w2_ref[...], preferred_element_type=jnp.float32) + b2_ref[...]
    o_ref[...] = 1.0 / (1.0 + jnp.exp(-z))


_TC_HEAD = pl.pallas_call(
    _tc6_body,
    out_shape=jax.ShapeDtypeStruct((G, 2), jnp.float32),
)


@jax.jit
def kernel(x, edge_index, batch, W1, b1, W2, b2, W3, b3, l1_W, l1_b, l2_W, l2_b):
    f32 = jnp.float32
    x_pad = jnp.zeros((NA, D_IN), f32).at[:N].set(x)
    W1p = jnp.zeros((D_IN, 64), f32).at[:, :42].set(W1)
    W1q = jnp.stack([W1p[:, 16 * q:16 * (q + 1)] for q in range(4)])
    b1p = jnp.zeros((1, 64), f32).at[0, :42].set(b1)
    W2p = jnp.zeros((64, 32), f32).at[:42, :24].set(W2)
    W2s = jnp.stack([W2p[:, :16], W2p[:, 16:]])
    b2p = jnp.zeros((1, 32), f32).at[0, :24].set(b2)
    W3p = jnp.zeros((32, 16), f32).at[:24].set(W3)
    W3s = jnp.stack([W3p[:, :8], W3p[:, 8:]])
    b3s = jnp.stack([b3[:8], b3[8:]]).reshape(2, 1, 8)

    src = edge_index[0].astype(jnp.int32)
    dst = edge_index[1].astype(jnp.int32)
    pad_e = jnp.full((EA - E,), N, jnp.int32)
    srcv = jnp.concatenate([src, pad_e])
    srcq_rows = jnp.concatenate(
        [srcv + q * NA for q in range(4)]).reshape(4 * ER, 128)
    dst_rows = jnp.concatenate([dst, pad_e]).reshape(ER, 128)
    bat_rows = jnp.concatenate(
        [batch.astype(jnp.int32), jnp.full((NBP - N,), G, jnp.int32)]
    ).reshape(BR, 128)
    poolv = jnp.concatenate(
        [jnp.arange(N, dtype=jnp.int32), jnp.full((NBP - N,), N, jnp.int32)])
    poolq_rows = jnp.concatenate([poolv, poolv + NA]).reshape(2 * BR, 128)

    deg2, cnt2 = _sc_degree_count()(dst_rows, bat_rows,
                            jnp.zeros((NA,), f32), jnp.zeros((GA,), f32))
    t1 = _TC_MM1(x_pad, W1q)

    d0 = deg2[:NA].reshape(NA, 1)
    d1 = deg2[NA:].reshape(NA, 1)
    tp1, dinv = _TC_SCALE(d0, d1, t1)

    tq1 = tp1.reshape(4 * NA, 16)
    z16 = jnp.zeros((NA, 16), f32)
    sA = _sc_gather_scatter(ER, 16, NA, STRIPE_N, 0, 8)(
        srcq_rows, dst_rows, tq1, z16)
    sB = _sc_gather_scatter(ER, 16, NA, STRIPE_N, 2, 8)(
        srcq_rows, dst_rows, tq1, z16)
    s1 = jnp.concatenate([sA, sB]).reshape(4, NA, 16)
    tp2, t2 = _TC_L1(s1, s1, s1, s1, t1, t1, t1, t1, dinv, b1p, W2s)

    s2 = _sc_gather_scatter(ER, 16, NA, STRIPE_N, 0, 8)(
        srcq_rows, dst_rows, tp2.reshape(2 * NA, 16), z16).reshape(2, NA, 16)
    tp3, t3 = _TC_L2(s2, s2, t2, t2, dinv, b2p, W3s)

    s3 = _sc_gather_scatter(ER, 8, NA, STRIPE_N, 0, 16)(
        srcq_rows, dst_rows, tp3.reshape(2 * NA, 8),
        jnp.zeros((NA, 8), f32)).reshape(2, NA, 8)
    hp = _TC_H3(s3, t3, dinv, b3s)

    ps = _sc_gather_scatter(BR, 8, GA, STRIPE_G, 0, 13)(
        poolq_rows, bat_rows, hp.reshape(2 * NA, 8),
        jnp.zeros((GA, 8), f32))
    pa = ps[:GA][:G]
    pb = ps[GA:][:G]
    c0 = cnt2[:GA][:G].reshape(G, 1)
    c1 = cnt2[GA:][:G].reshape(G, 1)
    return _TC_HEAD(pa, pb, c0, c1, l1_W, l1_b.reshape(1, 10),
                    l2_W, l2_b.reshape(1, 2))


# speedup vs baseline: 13.5915x; 1.0814x over previous
import functools

import jax
import jax.numpy as jnp
from jax import lax
from jax.experimental import pallas as pl
from jax.experimental.pallas import tpu as pltpu
from jax.experimental.pallas import tpu_sc as plsc

N = 50000
E = 800000
G = 512
D_IN = 100

STRIPE_N = 3136
NA = 16 * STRIPE_N
ER = 6400
EA = ER * 128
BR = 416
NBP = BR * 128
STRIPE_G = 40
GA = 16 * STRIPE_G


def _sc_mesh():
    return plsc.VectorSubcoreMesh(core_axis_name="c", subcore_axis_name="s",
                                  num_cores=2, num_subcores=16)


@functools.cache
def _sc_gather_scatter(n_rows, dh, nacc, stripe, qbase, kk):
    rpt = n_rows // 16
    n_sc = rpt // kk

    @functools.partial(
        pl.kernel,
        mesh=_sc_mesh(),
        compiler_params=pltpu.CompilerParams(use_tc_tiling_on_sc=False),
        out_type=jax.ShapeDtypeStruct((2 * nacc, dh), jnp.float32),
        scratch_types=[
            pltpu.VMEM((kk, 128), jnp.int32),
            pltpu.VMEM((kk, 128), jnp.int32),
            pltpu.VMEM((kk, 128, dh), jnp.float32),
            pltpu.VMEM((stripe, dh), jnp.float32),
            pltpu.VMEM_SHARED((nacc, dh), jnp.float32),
            pltpu.SemaphoreType.DMA,
            pltpu.SemaphoreType.DMA,
        ],
    )
    def k(gidx_hbm, sidx_hbm, table_hbm, zeros_hbm, out_hbm,
          gidx_v, sidx_v, rows_v, stripe_v, acc, sem, sem2):
        c = lax.axis_index("c")
        sid = lax.axis_index("s")
        pltpu.sync_copy(zeros_hbm.at[pl.ds(sid * stripe, stripe)], stripe_v)
        pltpu.sync_copy(stripe_v, acc.at[pl.ds(sid * stripe, stripe)])
        plsc.subcore_barrier()
        gbase = (qbase + c) * n_rows + sid * rpt
        base = sid * rpt

        def body(g, carry):
            pltpu.sync_copy(gidx_hbm.at[pl.ds(gbase + g * kk, kk)], gidx_v)
            pltpu.sync_copy(sidx_hbm.at[pl.ds(base + g * kk, kk)], sidx_v)
            descs = [pltpu.async_copy(table_hbm.at[gidx_v.at[j]],
                                      rows_v.at[j], sem)
                     for j in range(kk)]
            sdescs = []
            for j in range(kk):
                descs[j].wait()
                sdescs.append(pltpu.async_copy(
                    rows_v.at[j], acc.at[sidx_v.at[j]], sem2, add=True))
            for d in sdescs:
                d.wait()
            return carry

        lax.fori_loop(0, n_sc, body, 0)
        plsc.subcore_barrier()
        pltpu.sync_copy(acc.at[pl.ds(sid * stripe, stripe)], stripe_v)
        pltpu.sync_copy(stripe_v,
                        out_hbm.at[pl.ds(c * nacc + sid * stripe, stripe)])

    return k


@functools.cache
def _sc_degree_count():
    ept = ER // 32
    bpt = BR // 32

    @functools.partial(
        pl.kernel,
        mesh=_sc_mesh(),
        compiler_params=pltpu.CompilerParams(use_tc_tiling_on_sc=False),
        out_type=(jax.ShapeDtypeStruct((2 * NA,), jnp.float32),
                  jax.ShapeDtypeStruct((2 * GA,), jnp.float32)),
        scratch_types=[
            pltpu.VMEM((128,), jnp.int32),
            pltpu.VMEM((128,), jnp.float32),
            pltpu.VMEM((STRIPE_N,), jnp.float32),
            pltpu.VMEM((STRIPE_G,), jnp.float32),
            pltpu.VMEM_SHARED((NA,), jnp.float32),
            pltpu.VMEM_SHARED((GA,), jnp.float32),
        ],
    )
    def k(dst_hbm, bat_hbm, zn_hbm, zg_hbm, deg_hbm, cnt_hbm,
          idx_v, ones_v, sn_v, sg_v, dacc, cacc):
        c = lax.axis_index("c")
        sid = lax.axis_index("s")
        w = c * 16 + sid
        pltpu.sync_copy(zn_hbm.at[pl.ds(sid * STRIPE_N, STRIPE_N)], sn_v)
        pltpu.sync_copy(sn_v, dacc.at[pl.ds(sid * STRIPE_N, STRIPE_N)])
        pltpu.sync_copy(zg_hbm.at[pl.ds(sid * STRIPE_G, STRIPE_G)], sg_v)
        pltpu.sync_copy(sg_v, cacc.at[pl.ds(sid * STRIPE_G, STRIPE_G)])
        for j in range(8):
            ones_v[pl.ds(j * 16, 16)] = jnp.ones((16,), jnp.float32)
        plsc.subcore_barrier()

        ebase = w * ept

        def ebody(r, carry):
            pltpu.sync_copy(dst_hbm.at[ebase + r], idx_v)
            pltpu.sync_copy(ones_v, dacc.at[idx_v], add=True)
            return carry

        lax.fori_loop(0, ept, ebody, 0)

        bbase = w * bpt

        def bbody(r, carry):
            pltpu.sync_copy(bat_hbm.at[bbase + r], idx_v)
            pltpu.sync_copy(ones_v, cacc.at[idx_v], add=True)
            return carry

        lax.fori_loop(0, bpt, bbody, 0)
        plsc.subcore_barrier()
        pltpu.sync_copy(dacc.at[pl.ds(sid * STRIPE_N, STRIPE_N)], sn_v)
        pltpu.sync_copy(
            sn_v, deg_hbm.at[pl.ds(c * NA + sid * STRIPE_N, STRIPE_N)])
        pltpu.sync_copy(cacc.at[pl.ds(sid * STRIPE_G, STRIPE_G)], sg_v)
        pltpu.sync_copy(
            sg_v, cnt_hbm.at[pl.ds(c * GA + sid * STRIPE_G, STRIPE_G)])

    return k


def _mm1_body(x_ref, w_ref, o_ref):
    o_ref[0] = jnp.dot(x_ref[...], w_ref[0],
                       preferred_element_type=jnp.float32)


_TC_MM1 = pl.pallas_call(
    _mm1_body,
    grid=(4, 16),
    in_specs=[pl.BlockSpec((STRIPE_N, D_IN), lambda h, i: (i, 0)),
              pl.BlockSpec((1, D_IN, 16), lambda h, i: (h, 0, 0))],
    out_specs=pl.BlockSpec((1, STRIPE_N, 16), lambda h, i: (h, i, 0)),
    out_shape=jax.ShapeDtypeStruct((4, NA, 16), jnp.float32),
)


def _tc2_body(d0_ref, d1_ref, t1h_ref, tp_ref, dv_ref):
    dv = lax.rsqrt(d0_ref[...] + d1_ref[...] + 1.0)
    dv_ref[...] = dv
    tp_ref[0] = t1h_ref[0] * dv


_TC_SCALE = pl.pallas_call(
    _tc2_body,
    grid=(4, 16),
    in_specs=[pl.BlockSpec((STRIPE_N, 1), lambda h, i: (i, 0)),
              pl.BlockSpec((STRIPE_N, 1), lambda h, i: (i, 0)),
              pl.BlockSpec((1, STRIPE_N, 16), lambda h, i: (h, i, 0))],
    out_specs=[pl.BlockSpec((1, STRIPE_N, 16), lambda h, i: (h, i, 0)),
               pl.BlockSpec((STRIPE_N, 1), lambda h, i: (i, 0))],
    out_shape=[jax.ShapeDtypeStruct((4, NA, 16), jnp.float32),
               jax.ShapeDtypeStruct((NA, 1), jnp.float32)],
)


def _layer_body(*refs):
    n_s = (len(refs) - 5) // 2
    s_refs = refs[:n_s]
    t_refs = refs[n_s:2 * n_s]
    dv_ref, b_ref, w_ref, tp_ref, tr_ref = refs[2 * n_s:]
    dv = dv_ref[...]
    s = jnp.concatenate([r[0] for r in s_refs], axis=1)
    t = jnp.concatenate([r[0] for r in t_refs], axis=1)
    hcur = jnp.maximum(s * dv + t * (dv * dv) + b_ref[...], 0.0)
    th = jnp.dot(hcur, w_ref[0], preferred_element_type=jnp.float32)
    tr_ref[0] = th
    tp_ref[0] = th * dv


def _tc_layer(din, dout, n_parts):
    dh_in = din // n_parts
    dh_out = dout // 2
    part = lambda q: pl.BlockSpec((1, STRIPE_N, dh_in),
                                  lambda h, i, q=q: (q, i, 0))
    return pl.pallas_call(
        _layer_body,
        grid=(2, 16),
        in_specs=([part(q) for q in range(n_parts)]
                  + [part(q) for q in range(n_parts)]
                  + [pl.BlockSpec((STRIPE_N, 1), lambda h, i: (i, 0)),
                     pl.BlockSpec((1, din), lambda h, i: (0, 0)),
                     pl.BlockSpec((1, din, dh_out), lambda h, i: (h, 0, 0))]),
        out_specs=[pl.BlockSpec((1, STRIPE_N, dh_out), lambda h, i: (h, i, 0)),
                   pl.BlockSpec((1, STRIPE_N, dh_out), lambda h, i: (h, i, 0))],
        out_shape=[jax.ShapeDtypeStruct((2, NA, dh_out), jnp.float32),
                   jax.ShapeDtypeStruct((2, NA, dh_out), jnp.float32)],
    )


_TC_L1 = _tc_layer(64, 32, 4)
_TC_L2 = _tc_layer(32, 16, 2)


def _tc5_body(sh_ref, t3h_ref, dv_ref, b3h_ref, hp_ref):
    i = pl.program_id(1)
    dv = dv_ref[...]
    rid = i * STRIPE_N + lax.broadcasted_iota(jnp.int32, (STRIPE_N, 1), 0)
    valid = (rid < N).astype(jnp.float32)
    hp_ref[0] = (sh_ref[0] * dv + t3h_ref[0] * (dv * dv)
                 + b3h_ref[0]) * valid


_TC_H3 = pl.pallas_call(
    _tc5_body,
    grid=(2, 16),
    in_specs=[pl.BlockSpec((1, STRIPE_N, 8), lambda h, i: (h, i, 0)),
              pl.BlockSpec((1, STRIPE_N, 8), lambda h, i: (h, i, 0)),
              pl.BlockSpec((STRIPE_N, 1), lambda h, i: (i, 0)),
              pl.BlockSpec((1, 1, 8), lambda h, i: (h, 0, 0))],
    out_specs=pl.BlockSpec((1, STRIPE_N, 8), lambda h, i: (h, i, 0)),
    out_shape=jax.ShapeDtypeStruct((2, NA, 8), jnp.float32),
)


def _tc6_body(pa_ref, pb_ref, c0_ref, c1_ref, w1_ref, b1_ref, w2_ref, b2_ref,
              o_ref):
    cnt = jnp.maximum(c0_ref[...] + c1_ref[...], 1.0)
    p = jnp.concatenate([pa_ref[...], pb_ref[...]], axis=1) / cnt
    z = jnp.dot(p, w1_ref[...], preferred_element_type=jnp.float32) + b1_ref[...]
    z = jnp.dot(z, w2_ref[...], preferred_element_type=jnp.float32) + b2_ref[...]
    o_ref[...] = 1.0 / (1.0 + jnp.exp(-z))


_TC_HEAD = pl.pallas_call(
    _tc6_body,
    out_shape=jax.ShapeDtypeStruct((G, 2), jnp.float32),
)


@jax.jit
def kernel(x, edge_index, batch, W1, b1, W2, b2, W3, b3, l1_W, l1_b, l2_W, l2_b):
    f32 = jnp.float32
    x_pad = jnp.zeros((NA, D_IN), f32).at[:N].set(x)
    W1p = jnp.zeros((D_IN, 64), f32).at[:, :42].set(W1)
    W1q = jnp.stack([W1p[:, 16 * q:16 * (q + 1)] for q in range(4)])
    b1p = jnp.zeros((1, 64), f32).at[0, :42].set(b1)
    W2p = jnp.zeros((64, 32), f32).at[:42, :24].set(W2)
    W2s = jnp.stack([W2p[:, :16], W2p[:, 16:]])
    b2p = jnp.zeros((1, 32), f32).at[0, :24].set(b2)
    W3p = jnp.zeros((32, 16), f32).at[:24].set(W3)
    W3s = jnp.stack([W3p[:, :8], W3p[:, 8:]])
    b3s = jnp.stack([b3[:8], b3[8:]]).reshape(2, 1, 8)

    src = edge_index[0].astype(jnp.int32)
    dst = edge_index[1].astype(jnp.int32)
    pad_e = jnp.full((EA - E,), N, jnp.int32)
    srcv = jnp.concatenate([src, pad_e])
    srcq_rows = jnp.concatenate(
        [srcv + q * NA for q in range(4)]).reshape(4 * ER, 128)
    dst_rows = jnp.concatenate([dst, pad_e]).reshape(ER, 128)
    bat_rows = jnp.concatenate(
        [batch.astype(jnp.int32), jnp.full((NBP - N,), G, jnp.int32)]
    ).reshape(BR, 128)
    poolv = jnp.concatenate(
        [jnp.arange(N, dtype=jnp.int32), jnp.full((NBP - N,), N, jnp.int32)])
    poolq_rows = jnp.concatenate([poolv, poolv + NA]).reshape(2 * BR, 128)

    deg2, cnt2 = _sc_degree_count()(dst_rows, bat_rows,
                            jnp.zeros((NA,), f32), jnp.zeros((GA,), f32))
    t1 = _TC_MM1(x_pad, W1q)

    d0 = deg2[:NA].reshape(NA, 1)
    d1 = deg2[NA:].reshape(NA, 1)
    tp1, dinv = _TC_SCALE(d0, d1, t1)

    tq1 = tp1.reshape(4 * NA, 16)
    z16 = jnp.zeros((NA, 16), f32)
    sA = _sc_gather_scatter(ER, 16, NA, STRIPE_N, 0, 8)(
        srcq_rows, dst_rows, tq1, z16)
    sB = _sc_gather_scatter(ER, 16, NA, STRIPE_N, 2, 8)(
        srcq_rows, dst_rows, tq1, z16)
    s1 = jnp.concatenate([sA, sB]).reshape(4, NA, 16)
    tp2, t2 = _TC_L1(s1, s1, s1, s1, t1, t1, t1, t1, dinv, b1p, W2s)

    s2 = _sc_gather_scatter(ER, 16, NA, STRIPE_N, 0, 8)(
        srcq_rows, dst_rows, tp2.reshape(2 * NA, 16), z16).reshape(2, NA, 16)
    tp3, t3 = _TC_L2(s2, s2, t2, t2, dinv, b2p, W3s)

    s3 = _sc_gather_scatter(ER, 8, NA, STRIPE_N, 0, 16)(
        srcq_rows, dst_rows, tp3.reshape(2 * NA, 8),
        jnp.zeros((NA, 8), f32)).reshape(2, NA, 8)
    hp = _TC_H3(s3, t3, dinv, b3s)

    ps = _sc_gather_scatter(BR, 8, GA, STRIPE_G, 0, 13)(
        poolq_rows, bat_rows, hp.reshape(2 * NA, 8),
        jnp.zeros((GA, 8), f32))
    pa = ps[:GA][:G]
    pb = ps[GA:][:G]
    c0 = cnt2[:GA][:G].reshape(G, 1)
    c1 = cnt2[GA:][:G].reshape(G, 1)
    return _TC_HEAD(pa, pb, c0, c1, l1_W, l1_b.reshape(1, 10),
                    l2_W, l2_b.reshape(1, 2))


# speedup vs baseline: 13.8999x; 1.0227x over previous
import functools

import jax
import jax.numpy as jnp
from jax import lax
from jax.experimental import pallas as pl
from jax.experimental.pallas import tpu as pltpu
from jax.experimental.pallas import tpu_sc as plsc

N = 50000
E = 800000
G = 512
D_IN = 100

STRIPE_N = 3136
NA = 16 * STRIPE_N
ER = 6400
EA = ER * 128
BR = 416
NBP = BR * 128
STRIPE_G = 40
GA = 16 * STRIPE_G


def _sc_mesh():
    return plsc.VectorSubcoreMesh(core_axis_name="c", subcore_axis_name="s",
                                  num_cores=2, num_subcores=16)


@functools.cache
def _sc_gather_scatter(n_rows, dh, nacc, stripe, qbase, kk):
    rpt = n_rows // 16
    n_sc = rpt // kk

    @functools.partial(
        pl.kernel,
        mesh=_sc_mesh(),
        compiler_params=pltpu.CompilerParams(use_tc_tiling_on_sc=False),
        out_type=jax.ShapeDtypeStruct((2 * nacc, dh), jnp.float32),
        scratch_types=[
            pltpu.VMEM((kk, 128), jnp.int32),
            pltpu.VMEM((kk, 128), jnp.int32),
            pltpu.VMEM((kk, 128, dh), jnp.float32),
            pltpu.VMEM((stripe, dh), jnp.float32),
            pltpu.VMEM_SHARED((nacc, dh), jnp.float32),
            pltpu.SemaphoreType.DMA,
            pltpu.SemaphoreType.DMA,
        ],
    )
    def k(gidx_hbm, sidx_hbm, table_hbm, zeros_hbm, out_hbm,
          gidx_v, sidx_v, rows_v, stripe_v, acc, sem, sem2):
        c = lax.axis_index("c")
        sid = lax.axis_index("s")
        pltpu.sync_copy(zeros_hbm.at[pl.ds(sid * stripe, stripe)], stripe_v)
        pltpu.sync_copy(stripe_v, acc.at[pl.ds(sid * stripe, stripe)])
        plsc.subcore_barrier()
        gbase = (qbase + c) * n_rows + sid * rpt
        base = sid * rpt

        def body(g, carry):
            pltpu.sync_copy(gidx_hbm.at[pl.ds(gbase + g * kk, kk)], gidx_v)
            pltpu.sync_copy(sidx_hbm.at[pl.ds(base + g * kk, kk)], sidx_v)
            descs = [pltpu.async_copy(table_hbm.at[gidx_v.at[j]],
                                      rows_v.at[j], sem)
                     for j in range(kk)]
            sdescs = []
            for j in range(kk):
                descs[j].wait()
                sdescs.append(pltpu.async_copy(
                    rows_v.at[j], acc.at[sidx_v.at[j]], sem2, add=True))
            for d in sdescs:
                d.wait()
            return carry

        lax.fori_loop(0, n_sc, body, 0)
        plsc.subcore_barrier()
        pltpu.sync_copy(acc.at[pl.ds(sid * stripe, stripe)], stripe_v)
        pltpu.sync_copy(stripe_v,
                        out_hbm.at[pl.ds(c * nacc + sid * stripe, stripe)])

    return k


@functools.cache
def _sc_degree_count():
    ept = ER // 32
    bpt = BR // 32

    @functools.partial(
        pl.kernel,
        mesh=_sc_mesh(),
        compiler_params=pltpu.CompilerParams(use_tc_tiling_on_sc=False),
        out_type=(jax.ShapeDtypeStruct((2 * NA,), jnp.float32),
                  jax.ShapeDtypeStruct((2 * GA,), jnp.float32)),
        scratch_types=[
            pltpu.VMEM((25, 128), jnp.int32),
            pltpu.VMEM((128,), jnp.float32),
            pltpu.VMEM((STRIPE_N,), jnp.float32),
            pltpu.VMEM((STRIPE_G,), jnp.float32),
            pltpu.VMEM_SHARED((NA,), jnp.float32),
            pltpu.VMEM_SHARED((GA,), jnp.float32),
            pltpu.SemaphoreType.DMA,
        ],
    )
    def k(dst_hbm, bat_hbm, zn_hbm, zg_hbm, deg_hbm, cnt_hbm,
          idx_v, ones_v, sn_v, sg_v, dacc, cacc, sem):
        c = lax.axis_index("c")
        sid = lax.axis_index("s")
        w = c * 16 + sid
        pltpu.sync_copy(zn_hbm.at[pl.ds(sid * STRIPE_N, STRIPE_N)], sn_v)
        pltpu.sync_copy(sn_v, dacc.at[pl.ds(sid * STRIPE_N, STRIPE_N)])
        pltpu.sync_copy(zg_hbm.at[pl.ds(sid * STRIPE_G, STRIPE_G)], sg_v)
        pltpu.sync_copy(sg_v, cacc.at[pl.ds(sid * STRIPE_G, STRIPE_G)])
        for j in range(8):
            ones_v[pl.ds(j * 16, 16)] = jnp.ones((16,), jnp.float32)
        plsc.subcore_barrier()

        ebase = w * ept

        def ebody(g, carry):
            pltpu.sync_copy(dst_hbm.at[pl.ds(ebase + g * 25, 25)], idx_v)
            descs = [pltpu.async_copy(ones_v, dacc.at[idx_v.at[j]], sem,
                                      add=True)
                     for j in range(25)]
            for d in descs:
                d.wait()
            return carry

        lax.fori_loop(0, ept // 25, ebody, 0)

        bbase = w * bpt
        pltpu.sync_copy(bat_hbm.at[pl.ds(bbase, bpt)],
                        idx_v.at[pl.ds(0, bpt)])
        bdescs = [pltpu.async_copy(ones_v, cacc.at[idx_v.at[j]], sem,
                                   add=True)
                  for j in range(bpt)]
        for d in bdescs:
            d.wait()
        plsc.subcore_barrier()
        pltpu.sync_copy(dacc.at[pl.ds(sid * STRIPE_N, STRIPE_N)], sn_v)
        pltpu.sync_copy(
            sn_v, deg_hbm.at[pl.ds(c * NA + sid * STRIPE_N, STRIPE_N)])
        pltpu.sync_copy(cacc.at[pl.ds(sid * STRIPE_G, STRIPE_G)], sg_v)
        pltpu.sync_copy(
            sg_v, cnt_hbm.at[pl.ds(c * GA + sid * STRIPE_G, STRIPE_G)])

    return k


def _mm1_body(x_ref, w_ref, o_ref):
    o_ref[0] = jnp.dot(x_ref[...], w_ref[0],
                       preferred_element_type=jnp.float32)


_TC_MM1 = pl.pallas_call(
    _mm1_body,
    grid=(4, 16),
    in_specs=[pl.BlockSpec((STRIPE_N, D_IN), lambda h, i: (i, 0)),
              pl.BlockSpec((1, D_IN, 16), lambda h, i: (h, 0, 0))],
    out_specs=pl.BlockSpec((1, STRIPE_N, 16), lambda h, i: (h, i, 0)),
    out_shape=jax.ShapeDtypeStruct((4, NA, 16), jnp.float32),
)


def _tc2_body(d0_ref, d1_ref, t1h_ref, tp_ref, dv_ref):
    dv = lax.rsqrt(d0_ref[...] + d1_ref[...] + 1.0)
    dv_ref[...] = dv
    tp_ref[0] = t1h_ref[0] * dv


_TC_SCALE = pl.pallas_call(
    _tc2_body,
    grid=(4, 16),
    in_specs=[pl.BlockSpec((STRIPE_N, 1), lambda h, i: (i, 0)),
              pl.BlockSpec((STRIPE_N, 1), lambda h, i: (i, 0)),
              pl.BlockSpec((1, STRIPE_N, 16), lambda h, i: (h, i, 0))],
    out_specs=[pl.BlockSpec((1, STRIPE_N, 16), lambda h, i: (h, i, 0)),
               pl.BlockSpec((STRIPE_N, 1), lambda h, i: (i, 0))],
    out_shape=[jax.ShapeDtypeStruct((4, NA, 16), jnp.float32),
               jax.ShapeDtypeStruct((NA, 1), jnp.float32)],
)


def _layer_body(*refs):
    n_s = (len(refs) - 5) // 2
    s_refs = refs[:n_s]
    t_refs = refs[n_s:2 * n_s]
    dv_ref, b_ref, w_ref, tp_ref, tr_ref = refs[2 * n_s:]
    dv = dv_ref[...]
    s = jnp.concatenate([r[0] for r in s_refs], axis=1)
    t = jnp.concatenate([r[0] for r in t_refs], axis=1)
    hcur = jnp.maximum(s * dv + t * (dv * dv) + b_ref[...], 0.0)
    th = jnp.dot(hcur, w_ref[0], preferred_element_type=jnp.float32)
    tr_ref[0] = th
    tp_ref[0] = th * dv


def _tc_layer(din, dout, n_parts):
    dh_in = din // n_parts
    dh_out = dout // 2
    part = lambda q: pl.BlockSpec((1, STRIPE_N, dh_in),
                                  lambda h, i, q=q: (q, i, 0))
    return pl.pallas_call(
        _layer_body,
        grid=(2, 16),
        in_specs=([part(q) for q in range(n_parts)]
                  + [part(q) for q in range(n_parts)]
                  + [pl.BlockSpec((STRIPE_N, 1), lambda h, i: (i, 0)),
                     pl.BlockSpec((1, din), lambda h, i: (0, 0)),
                     pl.BlockSpec((1, din, dh_out), lambda h, i: (h, 0, 0))]),
        out_specs=[pl.BlockSpec((1, STRIPE_N, dh_out), lambda h, i: (h, i, 0)),
                   pl.BlockSpec((1, STRIPE_N, dh_out), lambda h, i: (h, i, 0))],
        out_shape=[jax.ShapeDtypeStruct((2, NA, dh_out), jnp.float32),
                   jax.ShapeDtypeStruct((2, NA, dh_out), jnp.float32)],
    )


_TC_L1 = _tc_layer(64, 32, 4)
_TC_L2 = _tc_layer(32, 16, 2)


def _tc5_body(sh_ref, t3h_ref, dv_ref, b3h_ref, hp_ref):
    i = pl.program_id(1)
    dv = dv_ref[...]
    rid = i * STRIPE_N + lax.broadcasted_iota(jnp.int32, (STRIPE_N, 1), 0)
    valid = (rid < N).astype(jnp.float32)
    hp_ref[0] = (sh_ref[0] * dv + t3h_ref[0] * (dv * dv)
                 + b3h_ref[0]) * valid


_TC_H3 = pl.pallas_call(
    _tc5_body,
    grid=(2, 16),
    in_specs=[pl.BlockSpec((1, STRIPE_N, 8), lambda h, i: (h, i, 0)),
              pl.BlockSpec((1, STRIPE_N, 8), lambda h, i: (h, i, 0)),
              pl.BlockSpec((STRIPE_N, 1), lambda h, i: (i, 0)),
              pl.BlockSpec((1, 1, 8), lambda h, i: (h, 0, 0))],
    out_specs=pl.BlockSpec((1, STRIPE_N, 8), lambda h, i: (h, i, 0)),
    out_shape=jax.ShapeDtypeStruct((2, NA, 8), jnp.float32),
)


def _tc6_body(pa_ref, pb_ref, c0_ref, c1_ref, w1_ref, b1_ref, w2_ref, b2_ref,
              o_ref):
    cnt = jnp.maximum(c0_ref[...] + c1_ref[...], 1.0)
    p = jnp.concatenate([pa_ref[...], pb_ref[...]], axis=1) / cnt
    z = jnp.dot(p, w1_ref[...], preferred_element_type=jnp.float32) + b1_ref[...]
    z = jnp.dot(z, w2_ref[...], preferred_element_type=jnp.float32) + b2_ref[...]
    o_ref[...] = 1.0 / (1.0 + jnp.exp(-z))


_TC_HEAD = pl.pallas_call(
    _tc6_body,
    out_shape=jax.ShapeDtypeStruct((G, 2), jnp.float32),
)


@jax.jit
def kernel(x, edge_index, batch, W1, b1, W2, b2, W3, b3, l1_W, l1_b, l2_W, l2_b):
    f32 = jnp.float32
    x_pad = jnp.zeros((NA, D_IN), f32).at[:N].set(x)
    W1p = jnp.zeros((D_IN, 64), f32).at[:, :42].set(W1)
    W1q = jnp.stack([W1p[:, 16 * q:16 * (q + 1)] for q in range(4)])
    b1p = jnp.zeros((1, 64), f32).at[0, :42].set(b1)
    W2p = jnp.zeros((64, 32), f32).at[:42, :24].set(W2)
    W2s = jnp.stack([W2p[:, :16], W2p[:, 16:]])
    b2p = jnp.zeros((1, 32), f32).at[0, :24].set(b2)
    W3p = jnp.zeros((32, 16), f32).at[:24].set(W3)
    W3s = jnp.stack([W3p[:, :8], W3p[:, 8:]])
    b3s = jnp.stack([b3[:8], b3[8:]]).reshape(2, 1, 8)

    src = edge_index[0].astype(jnp.int32)
    dst = edge_index[1].astype(jnp.int32)
    pad_e = jnp.full((EA - E,), N, jnp.int32)
    srcv = jnp.concatenate([src, pad_e])
    srcq_rows = jnp.concatenate(
        [srcv + q * NA for q in range(4)]).reshape(4 * ER, 128)
    dst_rows = jnp.concatenate([dst, pad_e]).reshape(ER, 128)
    bat_rows = jnp.concatenate(
        [batch.astype(jnp.int32), jnp.full((NBP - N,), G, jnp.int32)]
    ).reshape(BR, 128)
    poolv = jnp.concatenate(
        [jnp.arange(N, dtype=jnp.int32), jnp.full((NBP - N,), N, jnp.int32)])
    poolq_rows = jnp.concatenate([poolv, poolv + NA]).reshape(2 * BR, 128)

    deg2, cnt2 = _sc_degree_count()(dst_rows, bat_rows,
                            jnp.zeros((NA,), f32), jnp.zeros((GA,), f32))
    t1 = _TC_MM1(x_pad, W1q)

    d0 = deg2[:NA].reshape(NA, 1)
    d1 = deg2[NA:].reshape(NA, 1)
    tp1, dinv = _TC_SCALE(d0, d1, t1)

    tq1 = tp1.reshape(4 * NA, 16)
    z16 = jnp.zeros((NA, 16), f32)
    sA = _sc_gather_scatter(ER, 16, NA, STRIPE_N, 0, 8)(
        srcq_rows, dst_rows, tq1, z16)
    sB = _sc_gather_scatter(ER, 16, NA, STRIPE_N, 2, 8)(
        srcq_rows, dst_rows, tq1, z16)
    s1 = jnp.concatenate([sA, sB]).reshape(4, NA, 16)
    tp2, t2 = _TC_L1(s1, s1, s1, s1, t1, t1, t1, t1, dinv, b1p, W2s)

    s2 = _sc_gather_scatter(ER, 16, NA, STRIPE_N, 0, 8)(
        srcq_rows, dst_rows, tp2.reshape(2 * NA, 16), z16).reshape(2, NA, 16)
    tp3, t3 = _TC_L2(s2, s2, t2, t2, dinv, b2p, W3s)

    s3 = _sc_gather_scatter(ER, 8, NA, STRIPE_N, 0, 16)(
        srcq_rows, dst_rows, tp3.reshape(2 * NA, 8),
        jnp.zeros((NA, 8), f32)).reshape(2, NA, 8)
    hp = _TC_H3(s3, t3, dinv, b3s)

    ps = _sc_gather_scatter(BR, 8, GA, STRIPE_G, 0, 13)(
        poolq_rows, bat_rows, hp.reshape(2 * NA, 8),
        jnp.zeros((GA, 8), f32))
    pa = ps[:GA][:G]
    pb = ps[GA:][:G]
    c0 = cnt2[:GA][:G].reshape(G, 1)
    c1 = cnt2[GA:][:G].reshape(G, 1)
    return _TC_HEAD(pa, pb, c0, c1, l1_W, l1_b.reshape(1, 10),
                    l2_W, l2_b.reshape(1, 2))
